# baseline (speedup 1.0000x reference)
import functools

import jax
import jax.numpy as jnp
from jax.experimental import pallas as pl
from jax.experimental.pallas import tpu as pltpu
from jax.experimental.pallas import tpu_sc as plsc

B = 16384
NUM_DENSE = 13
NUM_SPARSE = 26
VOCAB = 100000
D = 64
NF = NUM_SPARSE + 1
NPAIR = NF * (NF - 1) // 2
INT_DIM = D + NPAIR

GATHER_WINDOW = 128
N_IDX = B * NUM_SPARSE

BB = 256


def _sc_gather(emb_flat, flat_idx):
    mesh = plsc.VectorSubcoreMesh(core_axis_name="core", subcore_axis_name="subcore")

    @functools.partial(
        pl.kernel,
        out_type=jax.ShapeDtypeStruct((N_IDX, D), jnp.float32),
        mesh=mesh,
        compiler_params=pltpu.CompilerParams(use_tc_tiling_on_sc=False),
    )
    def k(x_hbm, i_hbm, o_hbm):
        def body(i_vmem, o_vmem):
            pltpu.sync_copy(x_hbm.at[i_vmem.at[0]], o_vmem)

        pltpu.emit_pipeline(
            body,
            grid=(N_IDX // GATHER_WINDOW,),
            in_specs=[pl.BlockSpec((1, GATHER_WINDOW), lambda i: (0, i))],
            out_specs=[pl.BlockSpec((GATHER_WINDOW, D), lambda i: (i, 0))],
            core_axis_name=("core", "subcore"),
            dimension_semantics=(pltpu.PARALLEL,),
        )(i_hbm, o_hbm)

    return k(emb_flat, flat_idx)


def _tc_body(numT_ref, emb_ref,
             bw0T, bb0, bw1, bb1, bw2, bb2,
             tw0T, tb0, tw1, tb1, tw2, tb2, tw3, tb3, tw4, tb4,
             out_ref):
    f32 = jnp.float32
    xT = numT_ref[...]
    h = jnp.maximum(jnp.dot(bw0T[...], xT, preferred_element_type=f32) + bb0[...], 0.0)
    h = jnp.maximum(jnp.dot(bw1[...], h, preferred_element_type=f32) + bb1[...], 0.0)
    bot = jnp.maximum(jnp.dot(bw2[...], h, preferred_element_type=f32) + bb2[...], 0.0)

    embT = jnp.transpose(emb_ref[...], (0, 2, 1))
    T3 = jnp.concatenate([bot[None], embT], axis=0)
    zparts = []
    for i in range(1, NF):
        prod = T3[:i] * T3[i][None]
        zparts.append(jnp.sum(prod, axis=1))
    zcat = jnp.concatenate(zparts, axis=0)
    topT = jnp.concatenate([bot, zcat], axis=0)

    y = jnp.maximum(jnp.dot(tw0T[...], topT, preferred_element_type=f32) + tb0[...], 0.0)
    y = jnp.maximum(jnp.dot(tw1[...], y, preferred_element_type=f32) + tb1[...], 0.0)
    y = jnp.maximum(jnp.dot(tw2[...], y, preferred_element_type=f32) + tb2[...], 0.0)
    y = jnp.maximum(jnp.dot(tw3[...], y, preferred_element_type=f32) + tb3[...], 0.0)
    out_ref[...] = jnp.dot(tw4[...], y, preferred_element_type=f32) + tb4[...]


def _tc_forward(numT, emb_fm, weightsT, interpret=False):
    full = lambda a: pl.BlockSpec(a.shape, lambda b: tuple(0 for _ in a.shape))
    in_specs = [
        pl.BlockSpec((NUM_DENSE, BB), lambda b: (0, b)),
        pl.BlockSpec((NUM_SPARSE, BB, D), lambda b: (0, b, 0)),
    ] + [full(w) for w in weightsT]
    out = pl.pallas_call(
        _tc_body,
        grid=(B // BB,),
        in_specs=in_specs,
        out_specs=pl.BlockSpec((1, BB), lambda b: (0, b)),
        out_shape=jax.ShapeDtypeStruct((1, B), jnp.float32),
        interpret=interpret,
    )(numT, emb_fm, *weightsT)
    return out.reshape(B)


def kernel(numerical_input, categorical_input, emb_tables,
           bw0, bb0, bw1, bb1, bw2, bb2,
           tw0, tb0, tw1, tb1, tw2, tb2, tw3, tb3, tw4, tb4):
    emb_flat = emb_tables.reshape(NUM_SPARSE * VOCAB, D)
    offs = (jnp.arange(NUM_SPARSE, dtype=jnp.int32) * VOCAB)[:, None]
    flat_idx = (categorical_input.T.astype(jnp.int32) + offs).reshape(1, N_IDX)
    gathered = _sc_gather(emb_flat, flat_idx)
    emb_fm = gathered.reshape(NUM_SPARSE, B, D)

    numT = numerical_input.T
    col = lambda v: v.reshape(-1, 1)
    weightsT = [
        bw0.T, col(bb0), bw1.T, col(bb1), bw2.T, col(bb2),
        tw0.T, col(tb0), tw1.T, col(tb1), tw2.T, col(tb2),
        tw3.T, col(tb3), tw4.T, col(tb4),
    ]
    return _tc_forward(numT, emb_fm, weightsT)
